# baseline (speedup 1.0000x reference)
import functools

import jax
import jax.numpy as jnp
from jax import lax
from jax.experimental import pallas as pl
from jax.experimental.pallas import tpu as pltpu
from jax.experimental.pallas import tpu_sc as plsc

H = 16384
B = 16
S = 64
I = 256
NT = 16
RPT = H // NT
SUB = 128
NSUB = 4
CHUNK = SUB * NSUB
NSLOT = 4
DI = 2
DG = 1
TROWS = I + H

F32 = jnp.float32
I32 = jnp.int32


def _pack(rows, cols, vals):
    nnz = rows.shape[0]
    grain = NSLOT * CHUNK
    per_tile = -(-nnz // (NT * grain)) * grain
    pad = NT * per_tile - nnz
    r = jnp.pad(rows.astype(I32), (0, pad))
    c = jnp.pad(cols.astype(I32), (0, pad))
    v = jnp.pad(vals.astype(F32), (0, pad))
    nc = per_tile // CHUNK
    ishape = (NT, nc, NSUB, SUB)
    return r.reshape(ishape), c.reshape(ishape), v.reshape(NT, nc, CHUNK), nc


def _make_sc_kernel(nc_a, nc_h1):
    mesh = plsc.VectorSubcoreMesh(core_axis_name="c", subcore_axis_name="s",
                                  num_cores=1)

    @functools.partial(
        pl.kernel,
        out_type=(
            jax.ShapeDtypeStruct((TROWS, B), F32),
            jax.ShapeDtypeStruct((H, B), F32),
            jax.ShapeDtypeStruct((H, B), F32),
            jax.ShapeDtypeStruct((H, S, B), F32),
        ),
        mesh=mesh,
        compiler_params=pltpu.CompilerParams(use_tc_tiling_on_sc=False),
        scratch_types=[
            pltpu.VMEM_SHARED((H, B), F32),
            pltpu.VMEM((NSLOT, NSUB, SUB), I32),
            pltpu.VMEM((NSLOT, NSUB, SUB), I32),
            pltpu.VMEM((NSLOT, CHUNK), F32),
            pltpu.VMEM((NSLOT, CHUNK, B), F32),
            pltpu.VMEM((RPT, B), F32),
            pltpu.VMEM((RPT, B), F32),
            pltpu.VMEM((512, B), F32),
            pltpu.SemaphoreType.DMA((NSLOT,)),
            pltpu.SemaphoreType.DMA((NSLOT,)),
            pltpu.SemaphoreType.DMA((NSLOT,)),
            pltpu.SemaphoreType.DMA((2,)),
        ],
    )
    def rnn_sc(xg, ca, ra, va, ch1, rh1, vh1,
               tbl, h1, u, outs,
               acc, colb, rowb, valb, gbuf, q, pbuf, zbuf,
               semi, semg, sems, semw):
        w = lax.axis_index("s")
        row0 = w * RPT
        zero16 = jnp.zeros((B,), F32)

        def zrows(ref, n):
            def zb(i, carry):
                base = i * 16
                for l in range(16):
                    ref[base + l, :] = zero16
                return carry
            lax.fori_loop(0, n // 16, zb, 0)

        def spmm(colsR, rowsR, valsR, nc, table):
            def issue_idx(c, s):
                pltpu.async_copy(colsR.at[w, c], colb.at[s], semi.at[s])
                pltpu.async_copy(rowsR.at[w, c], rowb.at[s], semi.at[s])
                pltpu.async_copy(valsR.at[w, c], valb.at[s], semi.at[s])

            def wait_idx(c, s):
                pltpu.make_async_copy(colsR.at[w, c], colb.at[s], semi.at[s]).wait()
                pltpu.make_async_copy(rowsR.at[w, c], rowb.at[s], semi.at[s]).wait()
                pltpu.make_async_copy(valsR.at[w, c], valb.at[s], semi.at[s]).wait()

            def issue_gather(s):
                for j in range(NSUB):
                    pltpu.async_copy(table.at[colb.at[s, j]],
                                     gbuf.at[s, pl.ds(j * SUB, SUB)],
                                     semg.at[s])

            def wait_gather(s):
                for j in range(NSUB):
                    pltpu.make_async_copy(table.at[colb.at[s, j]],
                                          gbuf.at[s, pl.ds(j * SUB, SUB)],
                                          semg.at[s]).wait()

            def issue_scatter(s):
                for j in range(NSUB):
                    pltpu.async_copy(gbuf.at[s, pl.ds(j * SUB, SUB)],
                                     acc.at[rowb.at[s, j]], sems.at[s],
                                     add=True)

            def wait_scatter(s):
                for j in range(NSUB):
                    pltpu.make_async_copy(gbuf.at[s, pl.ds(j * SUB, SUB)],
                                          acc.at[rowb.at[s, j]],
                                          sems.at[s]).wait()

            def scale(s):
                def sb(g, carry):
                    base = g * 16
                    vv = valb[s, pl.ds(base, 16)]
                    for l in range(16):
                        k = base + l
                        gbuf[s, k, :] = gbuf[s, k, :] * vv[l]
                    return carry
                lax.fori_loop(0, CHUNK // 16, sb, 0)

            for k in range(DI):
                issue_idx(k, k)
            for k in range(DG):
                wait_idx(k, k)
                issue_gather(k)

            def group(g, carry):
                base = g * NSLOT
                for j in range(NSLOT):
                    c = base + j
                    si = (j + DI) % NSLOT
                    sg = (j + DG) % NSLOT

                    @pl.when(c + DI < nc)
                    def _(c=c, si=si):
                        @pl.when(c + DI >= NSLOT)
                        def _():
                            wait_scatter(si)
                        issue_idx(c + DI, si)

                    @pl.when(c + DG < nc)
                    def _(c=c, sg=sg):
                        wait_idx(c + DG, sg)
                        issue_gather(sg)

                    wait_gather(j)
                    scale(j)
                    issue_scatter(j)
                return carry
            lax.fori_loop(0, nc // NSLOT, group, 0)
            for s in range(NSLOT):
                wait_scatter(s)

        zrows(zbuf, 512)
        zrows(pbuf, RPT)
        pltpu.sync_copy(zbuf, acc.at[pl.ds(row0, 512)])
        pltpu.sync_copy(zbuf, acc.at[pl.ds(row0 + 512, 512)])
        pltpu.sync_copy(zbuf, tbl.at[pl.ds(I + row0, 512)])
        pltpu.sync_copy(zbuf, tbl.at[pl.ds(I + row0 + 512, 512)])
        pltpu.sync_copy(xg.at[pl.ds(w * 16, 16)], tbl.at[pl.ds(w * 16, 16)])
        plsc.subcore_barrier()

        def step(t, carry):
            spmm(ca, ra, va, nc_a, tbl)
            plsc.subcore_barrier()

            pltpu.sync_copy(acc.at[pl.ds(row0, RPT)], q)
            pltpu.sync_copy(zbuf, acc.at[pl.ds(row0, 512)])
            pltpu.sync_copy(zbuf, acc.at[pl.ds(row0 + 512, 512)])

            def fb(i, carry2):
                base = i * 16
                for l in range(16):
                    r = base + l
                    h0n = jnp.maximum(q[r, :], 0.0)
                    q[r, :] = h0n
                    pbuf[r, :] = h0n + pbuf[r, :]
                return carry2
            lax.fori_loop(0, RPT // 16, fb, 0)
            pltpu.async_copy(q, tbl.at[pl.ds(I + row0, RPT)], semw.at[0])
            pltpu.async_copy(pbuf, u.at[pl.ds(row0, RPT)], semw.at[1])
            pltpu.make_async_copy(q, tbl.at[pl.ds(I + row0, RPT)], semw.at[0]).wait()
            pltpu.make_async_copy(pbuf, u.at[pl.ds(row0, RPT)], semw.at[1]).wait()
            plsc.subcore_barrier()

            spmm(ch1, rh1, vh1, nc_h1, u)
            plsc.subcore_barrier()

            pltpu.sync_copy(acc.at[pl.ds(row0, RPT)], pbuf)
            pltpu.sync_copy(zbuf, acc.at[pl.ds(row0, 512)])
            pltpu.sync_copy(zbuf, acc.at[pl.ds(row0 + 512, 512)])

            def fd(i, carry2):
                base = i * 16
                for l in range(16):
                    r = base + l
                    pbuf[r, :] = jnp.maximum(pbuf[r, :], 0.0)
                return carry2
            lax.fori_loop(0, RPT // 16, fd, 0)
            pltpu.async_copy(pbuf, outs.at[pl.ds(row0, RPT), t], semw.at[0])
            @pl.when(t < S - 1)
            def _():
                pltpu.sync_copy(xg.at[pl.ds((t + 1) * I + w * 16, 16)],
                                tbl.at[pl.ds(w * 16, 16)])

            @pl.when(t == S - 1)
            def _():
                pltpu.sync_copy(pbuf, h1.at[pl.ds(row0, RPT)])
            pltpu.make_async_copy(pbuf, outs.at[pl.ds(row0, RPT), t],
                                  semw.at[0]).wait()
            plsc.subcore_barrier()
            return carry
        lax.fori_loop(0, S, step, 0)

    return rnn_sc


KBLK = 2048


def _tc_proj_body(w_ref, m_ref, b_ref, o_ref):
    k = pl.program_id(0)

    @pl.when(k == 0)
    def _():
        o_ref[...] = jnp.broadcast_to(b_ref[:, 0:1], o_ref.shape)
    o_ref[...] += jnp.dot(w_ref[...], m_ref[...],
                          preferred_element_type=F32)


def _tc_project(W_out, M, b2d):
    return pl.pallas_call(
        _tc_proj_body,
        grid=(H // KBLK,),
        in_specs=[
            pl.BlockSpec((I, KBLK), lambda k: (0, k)),
            pl.BlockSpec((KBLK, S * B), lambda k: (k, 0)),
            pl.BlockSpec((I, 128), lambda k: (0, 0)),
        ],
        out_specs=pl.BlockSpec((I, S * B), lambda k: (0, 0)),
        out_shape=jax.ShapeDtypeStruct((I, S * B), F32),
    )(W_out, M, b2d)


def kernel(x, rows_ih_0, cols_ih_0, vals_ih_0, rows_hh_0, cols_hh_0, vals_hh_0,
           rows_ih_1, cols_ih_1, vals_ih_1, rows_hh_1, cols_hh_1, vals_hh_1,
           W_out, b_out):
    xg = x.transpose(1, 2, 0).reshape(S * I, B)

    rows_a = jnp.concatenate([rows_ih_0.astype(I32), rows_hh_0.astype(I32)])
    cols_a = jnp.concatenate([cols_ih_0.astype(I32), cols_hh_0.astype(I32) + I])
    vals_a = jnp.concatenate([vals_ih_0, vals_hh_0])
    ra, ca, va, nc_a = _pack(rows_a, cols_a, vals_a)
    rh1, ch1, vh1, nc_h1 = _pack(rows_hh_1, cols_hh_1, vals_hh_1)

    rnn = _make_sc_kernel(nc_a, nc_h1)
    tbl, h1, _u, outs = rnn(xg, ca, ra, va, ch1, rh1, vh1)
    h0 = tbl[I:]

    b2d = jnp.broadcast_to(b_out.reshape(I, 1), (I, 128))
    out_mat = _tc_project(W_out, outs.reshape(H, S * B), b2d)

    out = out_mat.reshape(I, S, B).transpose(2, 1, 0)
    h_t = jnp.stack([h0.T, h1.T])
    return (out, h_t)


# speedup vs baseline: 1.0649x; 1.0649x over previous
import functools

import jax
import jax.numpy as jnp
from jax import lax
from jax.experimental import pallas as pl
from jax.experimental.pallas import tpu as pltpu
from jax.experimental.pallas import tpu_sc as plsc

H = 16384
B = 16
S = 64
I = 256
NT = 16
RPT = H // NT
SUB = 128
NSUB = 4
CHUNK = SUB * NSUB
NSLOT = 4
DI = 2
DG = 1
TROWS = I + H

F32 = jnp.float32
I32 = jnp.int32


def _pack(rows, cols, vals):
    nnz = rows.shape[0]
    npad = -(-(nnz + CHUNK) // (NSLOT * CHUNK)) * (NSLOT * CHUNK)
    r = jnp.pad(rows.astype(I32), (0, npad - nnz))
    c = jnp.pad(cols.astype(I32), (0, npad - nnz))
    v = jnp.pad(vals.astype(F32), (0, npad - nnz))
    bounds = jnp.searchsorted(
        rows.astype(I32), jnp.arange(NT + 1, dtype=I32) * RPT).astype(I32)
    lows, his = bounds[:NT], bounds[1:]
    ncmax = npad // CHUNK
    return c.reshape(npad // SUB, SUB), r, v, lows, his, ncmax


def _make_sc_kernel(ncmax_i, ncmax_0, ncmax_1):
    mesh = plsc.VectorSubcoreMesh(core_axis_name="c", subcore_axis_name="s",
                                  num_cores=1)

    @functools.partial(
        pl.kernel,
        out_type=(
            jax.ShapeDtypeStruct((TROWS, B), F32),
            jax.ShapeDtypeStruct((H, B), F32),
            jax.ShapeDtypeStruct((H, B), F32),
            jax.ShapeDtypeStruct((H, S, B), F32),
        ),
        mesh=mesh,
        compiler_params=pltpu.CompilerParams(use_tc_tiling_on_sc=False,
                                             needs_layout_passes=False),
        scratch_types=[
            pltpu.VMEM((6, B), I32),
            pltpu.VMEM((NSLOT, NSUB, SUB), I32),
            pltpu.VMEM((NSLOT, CHUNK), I32),
            pltpu.VMEM((NSLOT, CHUNK), F32),
            pltpu.VMEM((NSLOT, CHUNK, B), F32),
            pltpu.VMEM((RPT, B), F32),
            pltpu.VMEM((RPT, B), F32),
            pltpu.VMEM((RPT, B), F32),
            pltpu.VMEM((512, B), F32),
            pltpu.SemaphoreType.DMA((NSLOT,)),
            pltpu.SemaphoreType.DMA((NSLOT,)),
            pltpu.SemaphoreType.DMA((2,)),
        ],
    )
    def rnn_sc(xg, bnd, cih, rih, vih, ch0, rh0, vh0, ch1, rh1, vh1,
               tbl, h1, u, outs,
               bndv, colb, rowb, valb, gbuf, acc, q, pbuf, zbuf,
               semi, semg, semw):
        w = lax.axis_index("s")
        row0 = w * RPT
        wvec = jnp.full((B,), w, dtype=I32)
        zero16 = jnp.zeros((B,), F32)

        def zrows(ref, n):
            def zb(i, carry):
                base = i * 16
                for l in range(16):
                    ref[base + l, :] = zero16
                return carry
            lax.fori_loop(0, n // 16, zb, 0)

        def spmm(colsR, rowsR, valsR, ncmax, brow, table):
            lo = plsc.load_gather(bndv.at[brow], [wvec])[0]
            hi = plsc.load_gather(bndv.at[brow + 1], [wvec])[0]
            start = (lo // SUB) * SUB
            nch = (hi - start + CHUNK - 1) // CHUNK
            lov = jnp.full((B,), lo, dtype=I32)
            hiv = jnp.full((B,), hi, dtype=I32)

            def issue_idx(c, s):
                i0 = start // SUB + c * NSUB
                k0 = start + c * CHUNK
                pltpu.async_copy(colsR.at[pl.ds(i0, NSUB)], colb.at[s],
                                 semi.at[s])
                pltpu.async_copy(rowsR.at[pl.ds(k0, CHUNK)], rowb.at[s],
                                 semi.at[s])
                pltpu.async_copy(valsR.at[pl.ds(k0, CHUNK)], valb.at[s],
                                 semi.at[s])

            def wait_idx(c, s):
                i0 = start // SUB + c * NSUB
                k0 = start + c * CHUNK
                pltpu.make_async_copy(colsR.at[pl.ds(i0, NSUB)], colb.at[s],
                                      semi.at[s]).wait()
                pltpu.make_async_copy(rowsR.at[pl.ds(k0, CHUNK)], rowb.at[s],
                                      semi.at[s]).wait()
                pltpu.make_async_copy(valsR.at[pl.ds(k0, CHUNK)], valb.at[s],
                                      semi.at[s]).wait()

            def issue_gather(s):
                for j in range(NSUB):
                    pltpu.async_copy(table.at[colb.at[s, j]],
                                     gbuf.at[s, pl.ds(j * SUB, SUB)],
                                     semg.at[s])

            def wait_gather(s):
                for j in range(NSUB):
                    pltpu.make_async_copy(table.at[colb.at[s, j]],
                                          gbuf.at[s, pl.ds(j * SUB, SUB)],
                                          semg.at[s]).wait()

            def mac(c, s):
                kbase = start + c * CHUNK

                def sb(g, carry):
                    goff = g * 16
                    kv = kbase + goff + lax.iota(I32, 16)
                    vv = valb[s, pl.ds(goff, 16)]
                    rv = rowb[s, pl.ds(goff, 16)]
                    msk = (kv >= lov) & (kv < hiv)
                    vm = jnp.where(msk, vv, 0.0)
                    rl = jnp.clip(rv - row0, 0, RPT - 1)
                    for l in range(16):
                        plsc.addupdate(acc.at[rl[l]],
                                       gbuf[s, goff + l, :] * vm[l])
                    return carry
                lax.fori_loop(0, CHUNK // 16, sb, 0)

            for k in range(DI):
                @pl.when(k < nch)
                def _(k=k):
                    issue_idx(k, k)
            for k in range(DG):
                @pl.when(k < nch)
                def _(k=k):
                    wait_idx(k, k)
                    issue_gather(k)

            def group(gi, carry):
                base = gi * NSLOT
                for j in range(NSLOT):
                    c = base + j
                    si = (j + DI) % NSLOT
                    sg = (j + DG) % NSLOT

                    @pl.when(c + DI < nch)
                    def _(c=c, si=si):
                        issue_idx(c + DI, si)

                    @pl.when(c + DG < nch)
                    def _(c=c, sg=sg):
                        wait_idx(c + DG, sg)
                        issue_gather(sg)

                    @pl.when(c < nch)
                    def _(c=c, j=j):
                        wait_gather(j)
                        mac(c, j)
                return carry
            lax.fori_loop(0, ncmax // NSLOT, group, 0)

        pltpu.sync_copy(bnd, bndv)
        zrows(zbuf, 512)
        zrows(pbuf, RPT)
        zrows(acc, RPT)
        pltpu.sync_copy(zbuf, tbl.at[pl.ds(I + row0, 512)])
        pltpu.sync_copy(zbuf, tbl.at[pl.ds(I + row0 + 512, 512)])
        pltpu.sync_copy(xg.at[pl.ds(w * 16, 16)], tbl.at[pl.ds(w * 16, 16)])
        plsc.subcore_barrier()

        def step(t, carry):
            spmm(cih, rih, vih, ncmax_i, 0, tbl)
            spmm(ch0, rh0, vh0, ncmax_0, 2, tbl)

            def fb(i, carry2):
                base = i * 16
                for l in range(16):
                    r = base + l
                    h0n = jnp.maximum(acc[r, :], 0.0)
                    acc[r, :] = zero16
                    q[r, :] = h0n
                    pbuf[r, :] = h0n + pbuf[r, :]
                return carry2
            lax.fori_loop(0, RPT // 16, fb, 0)
            pltpu.async_copy(q, tbl.at[pl.ds(I + row0, RPT)], semw.at[0])
            pltpu.async_copy(pbuf, u.at[pl.ds(row0, RPT)], semw.at[1])
            pltpu.make_async_copy(q, tbl.at[pl.ds(I + row0, RPT)],
                                  semw.at[0]).wait()
            pltpu.make_async_copy(pbuf, u.at[pl.ds(row0, RPT)],
                                  semw.at[1]).wait()
            plsc.subcore_barrier()

            spmm(ch1, rh1, vh1, ncmax_1, 4, u)

            def fd(i, carry2):
                base = i * 16
                for l in range(16):
                    r = base + l
                    pbuf[r, :] = jnp.maximum(acc[r, :], 0.0)
                    acc[r, :] = zero16
                return carry2
            lax.fori_loop(0, RPT // 16, fd, 0)
            pltpu.async_copy(pbuf, outs.at[pl.ds(row0, RPT), t], semw.at[0])
            @pl.when(t < S - 1)
            def _():
                pltpu.sync_copy(xg.at[pl.ds((t + 1) * I + w * 16, 16)],
                                tbl.at[pl.ds(w * 16, 16)])

            @pl.when(t == S - 1)
            def _():
                pltpu.sync_copy(pbuf, h1.at[pl.ds(row0, RPT)])
            pltpu.make_async_copy(pbuf, outs.at[pl.ds(row0, RPT), t],
                                  semw.at[0]).wait()
            plsc.subcore_barrier()
            return carry
        lax.fori_loop(0, S, step, 0)

    return rnn_sc


KBLK = 2048


def _tc_proj_body(w_ref, m_ref, b_ref, o_ref):
    k = pl.program_id(0)

    @pl.when(k == 0)
    def _():
        o_ref[...] = jnp.broadcast_to(b_ref[:, 0:1], o_ref.shape)
    o_ref[...] += jnp.dot(w_ref[...], m_ref[...],
                          preferred_element_type=F32)


def _tc_project(W_out, M, b2d):
    return pl.pallas_call(
        _tc_proj_body,
        grid=(H // KBLK,),
        in_specs=[
            pl.BlockSpec((I, KBLK), lambda k: (0, k)),
            pl.BlockSpec((KBLK, S * B), lambda k: (k, 0)),
            pl.BlockSpec((I, 128), lambda k: (0, 0)),
        ],
        out_specs=pl.BlockSpec((I, S * B), lambda k: (0, 0)),
        out_shape=jax.ShapeDtypeStruct((I, S * B), F32),
    )(W_out, M, b2d)


def kernel(x, rows_ih_0, cols_ih_0, vals_ih_0, rows_hh_0, cols_hh_0, vals_hh_0,
           rows_ih_1, cols_ih_1, vals_ih_1, rows_hh_1, cols_hh_1, vals_hh_1,
           W_out, b_out):
    xg = x.transpose(1, 2, 0).reshape(S * I, B)

    cih, rih, vih, lo_i, hi_i, ncmax_i = _pack(
        rows_ih_0, cols_ih_0, vals_ih_0)
    ch0, rh0, vh0, lo_0, hi_0, ncmax_0 = _pack(
        rows_hh_0, cols_hh_0.astype(I32) + I, vals_hh_0)
    ch1, rh1, vh1, lo_1, hi_1, ncmax_1 = _pack(
        rows_hh_1, cols_hh_1, vals_hh_1)
    bnd = jnp.stack([lo_i, hi_i, lo_0, hi_0, lo_1, hi_1])

    rnn = _make_sc_kernel(ncmax_i, ncmax_0, ncmax_1)
    tbl, h1, _u, outs = rnn(xg, bnd, cih, rih, vih, ch0, rh0, vh0,
                            ch1, rh1, vh1)
    h0 = tbl[I:]

    b2d = jnp.broadcast_to(b_out.reshape(I, 1), (I, 128))
    out_mat = _tc_project(W_out, outs.reshape(H, S * B), b2d)

    out = out_mat.reshape(I, S, B).transpose(2, 1, 0)
    h_t = jnp.stack([h0.T, h1.T])
    return (out, h_t)


# speedup vs baseline: 1.0929x; 1.0263x over previous
import functools

import jax
import jax.numpy as jnp
from jax import lax
from jax.experimental import pallas as pl
from jax.experimental.pallas import tpu as pltpu
from jax.experimental.pallas import tpu_sc as plsc

H = 16384
B = 16
S = 64
I = 256
NT = 16
RPT = H // NT
SUB = 128
NSUB = 4
CHUNK = SUB * NSUB
NSLOT = 4
DI = 2
DG = 1
TROWS = I + H

F32 = jnp.float32
I32 = jnp.int32


def _pack(rows, cols, vals):
    nnz = rows.shape[0]
    npad = -(-(nnz + CHUNK) // (NSLOT * CHUNK)) * (NSLOT * CHUNK)
    r = jnp.pad(rows.astype(I32), (0, npad - nnz))
    c = jnp.pad(cols.astype(I32), (0, npad - nnz))
    v = jnp.pad(vals.astype(F32), (0, npad - nnz))
    bounds = jnp.searchsorted(
        rows.astype(I32), jnp.arange(NT + 1, dtype=I32) * RPT).astype(I32)
    lows, his = bounds[:NT], bounds[1:]
    ncmax = npad // CHUNK
    return c, r, v, lows, his, ncmax


def _make_sc_kernel(ncmax_i, ncmax_0, ncmax_1):
    mesh = plsc.VectorSubcoreMesh(core_axis_name="c", subcore_axis_name="s",
                                  num_cores=1)

    @functools.partial(
        pl.kernel,
        out_type=(
            jax.ShapeDtypeStruct((TROWS, B), F32),
            jax.ShapeDtypeStruct((H, B), F32),
            jax.ShapeDtypeStruct((H, B), F32),
            jax.ShapeDtypeStruct((H, S, B), F32),
        ),
        mesh=mesh,
        compiler_params=pltpu.CompilerParams(use_tc_tiling_on_sc=False,
                                             needs_layout_passes=False),
        scratch_types=[
            pltpu.VMEM((6, B), I32),
            pltpu.VMEM((NSLOT, CHUNK), I32),
            pltpu.VMEM((NSLOT, CHUNK), I32),
            pltpu.VMEM((NSLOT, CHUNK), F32),
            pltpu.VMEM((NSLOT, CHUNK, B), F32),
            pltpu.VMEM((RPT, B), F32),
            pltpu.VMEM((RPT, B), F32),
            pltpu.VMEM((RPT, B), F32),
            pltpu.VMEM((512, B), F32),
            pltpu.SemaphoreType.DMA((NSLOT,)),
            pltpu.SemaphoreType.DMA((NSLOT,)),
            pltpu.SemaphoreType.DMA((2,)),
        ],
    )
    def rnn_sc(xg, bnd, cih, rih, vih, ch0, rh0, vh0, ch1, rh1, vh1,
               tbl, h1, u, outs,
               bndv, colb, rowb, valb, gbuf, acc, q, pbuf, zbuf,
               semi, semg, semw):
        w = lax.axis_index("s")
        row0 = w * RPT
        wvec = jnp.full((B,), w, dtype=I32)
        zero16 = jnp.zeros((B,), F32)

        def zrows(ref, n):
            def zb(i, carry):
                base = i * 16
                for l in range(16):
                    ref[base + l, :] = zero16
                return carry
            lax.fori_loop(0, n // 16, zb, 0)

        def spmm(colsR, rowsR, valsR, ncmax, brow, table):
            lo = plsc.load_gather(bndv.at[brow], [wvec])[0]
            hi = plsc.load_gather(bndv.at[brow + 1], [wvec])[0]
            start = (lo // SUB) * SUB
            nch = (hi - start + CHUNK - 1) // CHUNK
            lov = jnp.full((B,), lo, dtype=I32)
            hiv = jnp.full((B,), hi, dtype=I32)

            def issue_idx(c, s):
                k0 = start + c * CHUNK
                pltpu.async_copy(colsR.at[pl.ds(k0, CHUNK)], colb.at[s],
                                 semi.at[s])
                pltpu.async_copy(rowsR.at[pl.ds(k0, CHUNK)], rowb.at[s],
                                 semi.at[s])
                pltpu.async_copy(valsR.at[pl.ds(k0, CHUNK)], valb.at[s],
                                 semi.at[s])

            def wait_idx(c, s):
                k0 = start + c * CHUNK
                pltpu.make_async_copy(colsR.at[pl.ds(k0, CHUNK)], colb.at[s],
                                      semi.at[s]).wait()
                pltpu.make_async_copy(rowsR.at[pl.ds(k0, CHUNK)], rowb.at[s],
                                      semi.at[s]).wait()
                pltpu.make_async_copy(valsR.at[pl.ds(k0, CHUNK)], valb.at[s],
                                      semi.at[s]).wait()

            def issue_gather(s):
                pltpu.async_copy(table.at[colb.at[s]], gbuf.at[s], semg.at[s])

            def wait_gather(s):
                pltpu.make_async_copy(table.at[colb.at[s]], gbuf.at[s],
                                      semg.at[s]).wait()

            def mac(c, s):
                kbase = start + c * CHUNK

                def sb(g, carry):
                    goff = g * 16
                    kv = kbase + goff + lax.iota(I32, 16)
                    vv = valb[s, pl.ds(goff, 16)]
                    rv = rowb[s, pl.ds(goff, 16)]
                    msk = (kv >= lov) & (kv < hiv)
                    vm = jnp.where(msk, vv, 0.0)
                    rl = jnp.clip(rv - row0, 0, RPT - 1)
                    for l in range(16):
                        plsc.addupdate(acc.at[rl[l]],
                                       gbuf[s, goff + l, :] * vm[l])
                    return carry
                lax.fori_loop(0, CHUNK // 16, sb, 0)

            for k in range(DI):
                @pl.when(k < nch)
                def _(k=k):
                    issue_idx(k, k)
            for k in range(DG):
                @pl.when(k < nch)
                def _(k=k):
                    wait_idx(k, k)
                    issue_gather(k)

            def group(gi, carry):
                base = gi * NSLOT
                for j in range(NSLOT):
                    c = base + j
                    si = (j + DI) % NSLOT
                    sg = (j + DG) % NSLOT

                    @pl.when(c + DI < nch)
                    def _(c=c, si=si):
                        issue_idx(c + DI, si)

                    @pl.when(c + DG < nch)
                    def _(c=c, sg=sg):
                        wait_idx(c + DG, sg)
                        issue_gather(sg)

                    @pl.when(c < nch)
                    def _(c=c, j=j):
                        wait_gather(j)
                        mac(c, j)
                return carry
            lax.fori_loop(0, ncmax // NSLOT, group, 0)

        pltpu.sync_copy(bnd, bndv)
        zrows(zbuf, 512)
        zrows(pbuf, RPT)
        zrows(acc, RPT)
        pltpu.sync_copy(zbuf, tbl.at[pl.ds(I + row0, 512)])
        pltpu.sync_copy(zbuf, tbl.at[pl.ds(I + row0 + 512, 512)])
        pltpu.sync_copy(xg.at[pl.ds(w * 16, 16)], tbl.at[pl.ds(w * 16, 16)])
        plsc.subcore_barrier()

        def step(t, carry):
            spmm(cih, rih, vih, ncmax_i, 0, tbl)
            spmm(ch0, rh0, vh0, ncmax_0, 2, tbl)

            def fb(i, carry2):
                base = i * 16
                for l in range(16):
                    r = base + l
                    h0n = jnp.maximum(acc[r, :], 0.0)
                    acc[r, :] = zero16
                    q[r, :] = h0n
                    pbuf[r, :] = h0n + pbuf[r, :]
                return carry2
            lax.fori_loop(0, RPT // 16, fb, 0)
            pltpu.async_copy(q, tbl.at[pl.ds(I + row0, RPT)], semw.at[0])
            pltpu.async_copy(pbuf, u.at[pl.ds(row0, RPT)], semw.at[1])
            pltpu.make_async_copy(q, tbl.at[pl.ds(I + row0, RPT)],
                                  semw.at[0]).wait()
            pltpu.make_async_copy(pbuf, u.at[pl.ds(row0, RPT)],
                                  semw.at[1]).wait()
            plsc.subcore_barrier()

            spmm(ch1, rh1, vh1, ncmax_1, 4, u)

            def fd(i, carry2):
                base = i * 16
                for l in range(16):
                    r = base + l
                    pbuf[r, :] = jnp.maximum(acc[r, :], 0.0)
                    acc[r, :] = zero16
                return carry2
            lax.fori_loop(0, RPT // 16, fd, 0)
            pltpu.async_copy(pbuf, outs.at[pl.ds(row0, RPT), t], semw.at[0])
            @pl.when(t < S - 1)
            def _():
                pltpu.sync_copy(xg.at[pl.ds((t + 1) * I + w * 16, 16)],
                                tbl.at[pl.ds(w * 16, 16)])

            @pl.when(t == S - 1)
            def _():
                pltpu.sync_copy(pbuf, h1.at[pl.ds(row0, RPT)])
            pltpu.make_async_copy(pbuf, outs.at[pl.ds(row0, RPT), t],
                                  semw.at[0]).wait()
            plsc.subcore_barrier()
            return carry
        lax.fori_loop(0, S, step, 0)

    return rnn_sc


KBLK = 2048


def _tc_proj_body(w_ref, m_ref, b_ref, o_ref):
    k = pl.program_id(0)

    @pl.when(k == 0)
    def _():
        o_ref[...] = jnp.broadcast_to(b_ref[:, 0:1], o_ref.shape)
    o_ref[...] += jnp.dot(w_ref[...], m_ref[...],
                          preferred_element_type=F32)


def _tc_project(W_out, M, b2d):
    return pl.pallas_call(
        _tc_proj_body,
        grid=(H // KBLK,),
        in_specs=[
            pl.BlockSpec((I, KBLK), lambda k: (0, k)),
            pl.BlockSpec((KBLK, S * B), lambda k: (k, 0)),
            pl.BlockSpec((I, 128), lambda k: (0, 0)),
        ],
        out_specs=pl.BlockSpec((I, S * B), lambda k: (0, 0)),
        out_shape=jax.ShapeDtypeStruct((I, S * B), F32),
    )(W_out, M, b2d)


def kernel(x, rows_ih_0, cols_ih_0, vals_ih_0, rows_hh_0, cols_hh_0, vals_hh_0,
           rows_ih_1, cols_ih_1, vals_ih_1, rows_hh_1, cols_hh_1, vals_hh_1,
           W_out, b_out):
    xg = x.transpose(1, 2, 0).reshape(S * I, B)

    cih, rih, vih, lo_i, hi_i, ncmax_i = _pack(
        rows_ih_0, cols_ih_0, vals_ih_0)
    ch0, rh0, vh0, lo_0, hi_0, ncmax_0 = _pack(
        rows_hh_0, cols_hh_0.astype(I32) + I, vals_hh_0)
    ch1, rh1, vh1, lo_1, hi_1, ncmax_1 = _pack(
        rows_hh_1, cols_hh_1, vals_hh_1)
    bnd = jnp.stack([lo_i, hi_i, lo_0, hi_0, lo_1, hi_1])

    rnn = _make_sc_kernel(ncmax_i, ncmax_0, ncmax_1)
    tbl, h1, _u, outs = rnn(xg, bnd, cih, rih, vih, ch0, rh0, vh0,
                            ch1, rh1, vh1)
    h0 = tbl[I:]

    b2d = jnp.broadcast_to(b_out.reshape(I, 1), (I, 128))
    out_mat = _tc_project(W_out, outs.reshape(H, S * B), b2d)

    out = out_mat.reshape(I, S, B).transpose(2, 1, 0)
    h_t = jnp.stack([h0.T, h1.T])
    return (out, h_t)


# speedup vs baseline: 1.1994x; 1.0975x over previous
import functools

import jax
import jax.numpy as jnp
from jax import lax
from jax.experimental import pallas as pl
from jax.experimental.pallas import tpu as pltpu
from jax.experimental.pallas import tpu_sc as plsc

H = 16384
B = 16
S = 64
I = 256
NT = 16
RPT = H // NT
CHUNK = 128
QUAD = 4 * CHUNK

F32 = jnp.float32
I32 = jnp.int32


def _pack(rows, cols, vals):
    nnz = rows.shape[0]
    per_tile = -(-nnz // (NT * QUAD)) * QUAD
    pad = NT * per_tile - nnz
    r = jnp.pad(rows.astype(I32), (0, pad))
    c = jnp.pad(cols.astype(I32), (0, pad))
    v = jnp.pad(vals.astype(F32), (0, pad))
    nc = per_tile // CHUNK
    shape = (NT, nc, CHUNK)
    return r.reshape(shape), c.reshape(shape), v.reshape(shape), nc


def _make_sc_kernel(nc_ih, nc_h0, nc_h1):
    mesh = plsc.VectorSubcoreMesh(core_axis_name="c", subcore_axis_name="s",
                                  num_cores=1)

    @functools.partial(
        pl.kernel,
        out_type=(
            jax.ShapeDtypeStruct((H, B), F32),
            jax.ShapeDtypeStruct((H, B), F32),
            jax.ShapeDtypeStruct((H, B), F32),
            jax.ShapeDtypeStruct((H, S, B), F32),
        ),
        mesh=mesh,
        compiler_params=pltpu.CompilerParams(use_tc_tiling_on_sc=False),
        scratch_types=[
            pltpu.VMEM_SHARED((H, B), F32),
            pltpu.VMEM((4, CHUNK), I32),
            pltpu.VMEM((4, CHUNK), I32),
            pltpu.VMEM((4, CHUNK), F32),
            pltpu.VMEM((4, CHUNK, B), F32),
            pltpu.VMEM((RPT, B), F32),
            pltpu.VMEM((RPT, B), F32),
            pltpu.VMEM((512, B), F32),
            pltpu.SemaphoreType.DMA((4,)),
            pltpu.SemaphoreType.DMA((4,)),
            pltpu.SemaphoreType.DMA((4,)),
        ],
    )
    def rnn_sc(xg, cih, rih, vih, ch0, rh0, vh0, ch1, rh1, vh1,
               h0, h1, u, outs,
               acc, colb, rowb, valb, gbuf, q, pbuf, zbuf,
               semi, semg, sems):
        w = lax.axis_index("s")
        row0 = w * RPT
        zero16 = jnp.zeros((B,), F32)

        def zrows(ref, n):
            def zb(i, carry):
                base = i * 16
                for l in range(16):
                    ref[base + l, :] = zero16
                return carry
            lax.fori_loop(0, n // 16, zb, 0)

        def spmm(colsR, rowsR, valsR, nc, table):
            def issue_idx(c, s):
                pltpu.async_copy(colsR.at[w, c], colb.at[s], semi.at[s])
                pltpu.async_copy(rowsR.at[w, c], rowb.at[s], semi.at[s])
                pltpu.async_copy(valsR.at[w, c], valb.at[s], semi.at[s])

            def wait_idx(c, s):
                pltpu.make_async_copy(colsR.at[w, c], colb.at[s], semi.at[s]).wait()
                pltpu.make_async_copy(rowsR.at[w, c], rowb.at[s], semi.at[s]).wait()
                pltpu.make_async_copy(valsR.at[w, c], valb.at[s], semi.at[s]).wait()

            def issue_gather(s):
                pltpu.async_copy(table.at[colb.at[s]], gbuf.at[s], semg.at[s])

            def wait_gather(s):
                pltpu.make_async_copy(table.at[colb.at[s]], gbuf.at[s],
                                      semg.at[s]).wait()

            def issue_scatter(s):
                pltpu.async_copy(gbuf.at[s], acc.at[rowb.at[s]], sems.at[s],
                                 add=True)

            def wait_scatter(s):
                pltpu.make_async_copy(gbuf.at[s], acc.at[rowb.at[s]],
                                      sems.at[s]).wait()

            def scale(s):
                def sb(g, carry):
                    base = g * 16
                    vv = valb[s, pl.ds(base, 16)]
                    for l in range(16):
                        k = base + l
                        gbuf[s, k, :] = gbuf[s, k, :] * vv[l]
                    return carry
                lax.fori_loop(0, CHUNK // 16, sb, 0)

            def do_chunk(c, s, s1, s2):
                @pl.when(c + 2 < nc)
                def _():
                    @pl.when(c >= 2)
                    def _():
                        wait_scatter(s2)
                    issue_idx(c + 2, s2)
                @pl.when(c + 1 < nc)
                def _():
                    wait_idx(c + 1, s1)
                    issue_gather(s1)
                wait_gather(s)
                scale(s)
                issue_scatter(s)

            issue_idx(0, 0)
            issue_idx(1, 1)
            wait_idx(0, 0)
            issue_gather(0)

            def quad(qi, carry):
                c0 = qi * 4
                do_chunk(c0 + 0, 0, 1, 2)
                do_chunk(c0 + 1, 1, 2, 3)
                do_chunk(c0 + 2, 2, 3, 0)
                do_chunk(c0 + 3, 3, 0, 1)
                return carry
            lax.fori_loop(0, nc // 4, quad, 0)
            for s in range(4):
                wait_scatter(s)

        zrows(zbuf, 512)
        zrows(pbuf, RPT)
        pltpu.sync_copy(zbuf, acc.at[pl.ds(row0, 512)])
        pltpu.sync_copy(zbuf, acc.at[pl.ds(row0 + 512, 512)])
        pltpu.sync_copy(zbuf, h0.at[pl.ds(row0, 512)])
        pltpu.sync_copy(zbuf, h0.at[pl.ds(row0 + 512, 512)])
        plsc.subcore_barrier()

        def step(t, carry):
            spmm(cih.at[t], rih, vih, nc_ih, xg)
            spmm(ch0, rh0, vh0, nc_h0, h0)
            plsc.subcore_barrier()

            pltpu.sync_copy(acc.at[pl.ds(row0, RPT)], q)
            pltpu.sync_copy(zbuf, acc.at[pl.ds(row0, 512)])
            pltpu.sync_copy(zbuf, acc.at[pl.ds(row0 + 512, 512)])

            def fb(i, carry2):
                base = i * 16
                for l in range(16):
                    r = base + l
                    h0n = jnp.maximum(q[r, :], 0.0)
                    q[r, :] = h0n
                    pbuf[r, :] = h0n + pbuf[r, :]
                return carry2
            lax.fori_loop(0, RPT // 16, fb, 0)
            pltpu.sync_copy(q, h0.at[pl.ds(row0, RPT)])
            pltpu.sync_copy(pbuf, u.at[pl.ds(row0, RPT)])
            plsc.subcore_barrier()

            spmm(ch1, rh1, vh1, nc_h1, u)
            plsc.subcore_barrier()

            pltpu.sync_copy(acc.at[pl.ds(row0, RPT)], pbuf)
            pltpu.sync_copy(zbuf, acc.at[pl.ds(row0, 512)])
            pltpu.sync_copy(zbuf, acc.at[pl.ds(row0 + 512, 512)])

            def fd(i, carry2):
                base = i * 16
                for l in range(16):
                    r = base + l
                    pbuf[r, :] = jnp.maximum(pbuf[r, :], 0.0)
                return carry2
            lax.fori_loop(0, RPT // 16, fd, 0)
            pltpu.sync_copy(pbuf, outs.at[pl.ds(row0, RPT), t])

            @pl.when(t == S - 1)
            def _():
                pltpu.sync_copy(pbuf, h1.at[pl.ds(row0, RPT)])
            plsc.subcore_barrier()
            return carry
        lax.fori_loop(0, S, step, 0)

    return rnn_sc


KBLK = 2048


def _tc_proj_body(w_ref, m_ref, b_ref, o_ref):
    k = pl.program_id(0)

    @pl.when(k == 0)
    def _():
        o_ref[...] = jnp.broadcast_to(b_ref[:, 0:1], o_ref.shape)
    o_ref[...] += jnp.dot(w_ref[...], m_ref[...],
                          preferred_element_type=F32)


def _tc_project(W_out, M, b2d):
    return pl.pallas_call(
        _tc_proj_body,
        grid=(H // KBLK,),
        in_specs=[
            pl.BlockSpec((I, KBLK), lambda k: (0, k)),
            pl.BlockSpec((KBLK, S * B), lambda k: (k, 0)),
            pl.BlockSpec((I, 128), lambda k: (0, 0)),
        ],
        out_specs=pl.BlockSpec((I, S * B), lambda k: (0, 0)),
        out_shape=jax.ShapeDtypeStruct((I, S * B), F32),
    )(W_out, M, b2d)


def kernel(x, rows_ih_0, cols_ih_0, vals_ih_0, rows_hh_0, cols_hh_0, vals_hh_0,
           rows_ih_1, cols_ih_1, vals_ih_1, rows_hh_1, cols_hh_1, vals_hh_1,
           W_out, b_out):
    xg = x.transpose(1, 2, 0).reshape(S * I, B)

    rih, cih, vih, nc_ih = _pack(rows_ih_0, cols_ih_0, vals_ih_0)
    rh0, ch0, vh0, nc_h0 = _pack(rows_hh_0, cols_hh_0, vals_hh_0)
    rh1, ch1, vh1, nc_h1 = _pack(rows_hh_1, cols_hh_1, vals_hh_1)

    shifts = (jnp.arange(S, dtype=I32) * I).reshape(S, 1, 1, 1)
    cih_t = cih[None] + shifts

    rnn = _make_sc_kernel(nc_ih, nc_h0, nc_h1)
    h0, h1, _u, outs = rnn(xg, cih_t, rih, vih, ch0, rh0, vh0,
                           ch1, rh1, vh1)

    b2d = jnp.broadcast_to(b_out.reshape(I, 1), (I, 128))
    out_mat = _tc_project(W_out, outs.reshape(H, S * B), b2d)

    out = out_mat.reshape(I, S, B).transpose(2, 1, 0)
    h_t = jnp.stack([h0.T, h1.T])
    return (out, h_t)
